# baseline (speedup 1.0000x reference)
import functools

import jax
import jax.numpy as jnp
from jax import lax
from jax.experimental import pallas as pl
from jax.experimental.pallas import tpu as pltpu
from jax.experimental.pallas import tpu_sc as plsc

_HID = 64
_NREG = _HID // 16
_NC, _NS = 2, 16
_NW = _NC * _NS
_C = 256
_IG = 128
_G = _C // _IG
_NBUF = 3


def _rsqrt16(x):
    i = lax.bitcast_convert_type(x, jnp.int32)
    y = lax.bitcast_convert_type(jnp.int32(0x5F3759DF) - (i >> 1), jnp.float32)
    hx = x * 0.5
    for _ in range(2):
        y = y * (1.5 - hx * y * y)
    return y


def _make_body(row0, n_rows, seq_len, eps):
    rows_per_w = n_rows // _NW
    n_chunks = rows_per_w // _C
    assert rows_per_w % _C == 0 and _C % 2 == 0
    n_trips = n_chunks // _NBUF
    n_tail = n_chunks - n_trips * _NBUF
    if n_tail < 2:
        n_trips -= 1
        n_tail += _NBUF
    assert n_trips >= 1 and n_tail >= 2

    def body(idx_hbm, word_hbm, pos_hbm, gamma_hbm, beta_hbm, out_hbm,
             idx0, idx1, idx2, rows0, rows1, rows2, sb0, sb1, sb2,
             pos_v, gamma_v, beta_v,
             gsem0, gsem1, gsem2, ssem0, ssem1, ssem2, isem0, isem1, isem2):
        idx_b = (idx0, idx1, idx2)
        rows_b = (rows0, rows1, rows2)
        sb_b = (sb0, sb1, sb2)
        gsem_b = (gsem0, gsem1, gsem2)
        ssem_b = (ssem0, ssem1, ssem2)
        isem_b = (isem0, isem1, isem2)

        wid = lax.axis_index("s") * _NC + lax.axis_index("c")
        base = wid * rows_per_w
        pltpu.sync_copy(pos_hbm, pos_v)
        pltpu.sync_copy(gamma_hbm, gamma_v)
        pltpu.sync_copy(beta_hbm, beta_v)
        g_regs = [gamma_v[pl.ds(16 * k, 16)] for k in range(_NREG)]
        b_regs = [beta_v[pl.ds(16 * k, 16)] for k in range(_NREG)]
        lanes = lax.iota(jnp.int32, 16)
        perms = [lanes ^ d for d in (1, 2, 4, 8)]

        def fire_idx(slot, g):
            cbase = row0 + base + g * _C
            pltpu.async_copy(idx_hbm.at[pl.ds(lax.div(cbase, _IG), _G)],
                             idx_b[slot], isem_b[slot])

        def wait_idx(slot):
            pltpu.make_async_copy(idx_hbm.at[pl.ds(0, _G)], idx_b[slot],
                                  isem_b[slot]).wait()

        def fire_gather(buf, g):
            for j in range(_G):
                pltpu.async_copy(word_hbm.at[idx_b[buf].at[j]],
                                 rows_b[buf].at[pl.ds(j * _IG, _IG)],
                                 gsem_b[buf])

        def wait_gather(buf):
            pltpu.make_async_copy(word_hbm.at[pl.ds(0, _C)], rows_b[buf],
                                  gsem_b[buf]).wait()

        def fire_store(buf, g):
            s128 = lax.div(base + g * _C, 2)
            pltpu.async_copy(sb_b[buf], out_hbm.at[pl.ds(s128, _C // 2)],
                             ssem_b[buf])

        def wait_store(buf):
            pltpu.make_async_copy(sb_b[buf], out_hbm.at[pl.ds(0, _C // 2)],
                                  ssem_b[buf]).wait()

        def compute(buf, cbase):
            rows_v = rows_b[buf]
            sb_v = sb_b[buf]
            off = lax.rem(row0 + cbase, seq_len)

            @plsc.parallel_loop(0, _C, unroll=4)
            def row_body(r):
                p = r + off
                for _ in range((_C + seq_len - 1) // seq_len + 1):
                    p = jnp.where(p >= seq_len, p - seq_len, p)
                q = r >> 1
                h0 = (r & 1) * _HID
                s = [rows_v[r, pl.ds(16 * k, 16)]
                     + pos_v[p, pl.ds(16 * k, 16)] for k in range(_NREG)]
                t = (s[0] + s[1]) + (s[2] + s[3])
                u = (s[0] * s[0] + s[1] * s[1]) + (s[2] * s[2] + s[3] * s[3])
                for perm in perms:
                    t = t + t.at[perm].get(mode="promise_in_bounds")
                    u = u + u.at[perm].get(mode="promise_in_bounds")
                m = t * (1.0 / _HID)
                var = jnp.maximum(u * (1.0 / _HID) - m * m, 0.0) + eps
                rs = _rsqrt16(var)
                for k in range(_NREG):
                    y = (s[k] - m) * rs
                    sb_v[q, pl.ds(h0 + 16 * k, 16)] = (y * g_regs[k]
                                                       + b_regs[k])

        fire_idx(0, 0)
        wait_idx(0)
        fire_gather(0, 0)
        fire_idx(1, 1)

        def trip_body(tt, carry):
            c0 = _NBUF * tt
            for par in range(_NBUF):
                nb = (par + 1) % _NBUF
                ns = (par + 2) % _NBUF
                wait_idx(nb)
                fire_gather(nb, c0 + par + 1)
                fire_idx(ns, c0 + par + 2)
                wait_gather(par)
                @pl.when(c0 + par >= _NBUF)
                def _():
                    wait_store(par)
                compute(par, base + (c0 + par) * _C)
                fire_store(par, c0 + par)
            return carry

        lax.fori_loop(0, n_trips, trip_body, 0)

        for c in range(n_trips * _NBUF, n_chunks):
            buf = c % _NBUF
            nb = (c + 1) % _NBUF
            if c + 1 < n_chunks:
                wait_idx(nb)
                fire_gather(nb, c + 1)
            if c + 2 < n_chunks:
                fire_idx((c + 2) % _NBUF, c + 2)
            wait_gather(buf)
            if c >= _NBUF:
                wait_store(buf)
            compute(buf, base + c * _C)
            fire_store(buf, c)

        for c in range(n_chunks - _NBUF, n_chunks):
            wait_store(c % _NBUF)

    return body


_NSPLIT = 2


def kernel(input_id, word_table, pos_table, gamma, beta):
    b, seq_len = input_id.shape
    n_rows = b * seq_len
    idx2d = input_id.reshape(n_rows // _IG, _IG).astype(jnp.int32)
    pos = pos_table[:seq_len]
    mesh = plsc.VectorSubcoreMesh(core_axis_name="c", subcore_axis_name="s")
    n_half = n_rows // _NSPLIT
    b_half = b // _NSPLIT
    parts = []
    for h in range(_NSPLIT):
        fn = functools.partial(
            pl.kernel,
            mesh=mesh,
            compiler_params=pltpu.CompilerParams(use_tc_tiling_on_sc=False),
            out_type=jax.ShapeDtypeStruct((n_half // 2, 2 * _HID),
                                          jnp.float32),
            scratch_types=(
                [pltpu.VMEM((_G, _IG), jnp.int32)] * _NBUF
                + [pltpu.VMEM((_C, _HID), jnp.float32)] * _NBUF
                + [pltpu.VMEM((_C // 2, 2 * _HID), jnp.float32)] * _NBUF
                + [pltpu.VMEM((seq_len, _HID), jnp.float32),
                   pltpu.VMEM((_HID,), jnp.float32),
                   pltpu.VMEM((_HID,), jnp.float32)]
                + [pltpu.SemaphoreType.DMA] * (3 * _NBUF)
            ),
            name=f"emb_ln_part{h}",
        )(_make_body(h * n_half, n_half, seq_len, 1e-12))
        parts.append(fn(idx2d, word_table, pos, gamma, beta)
                     .reshape(b_half, seq_len, _HID))
    return jnp.concatenate(parts, axis=0)


# speedup vs baseline: 1.1613x; 1.1613x over previous
import functools

import jax
import jax.numpy as jnp
from jax import lax
from jax.experimental import pallas as pl
from jax.experimental.pallas import tpu as pltpu
from jax.experimental.pallas import tpu_sc as plsc

_HID = 64
_NREG = _HID // 16
_NC, _NS = 2, 16
_NW = _NC * _NS
_C = 256
_IG = 128
_G = _C // _IG
_NBUF = 3


def _rsqrt16(x):
    i = lax.bitcast_convert_type(x, jnp.int32)
    y = lax.bitcast_convert_type(jnp.int32(0x5F3759DF) - (i >> 1), jnp.float32)
    hx = x * 0.5
    for _ in range(2):
        y = y * (1.5 - hx * y * y)
    return y


def _make_body(row0, n_rows, seq_len, eps):
    rows_per_w = n_rows // _NW
    n_chunks = rows_per_w // _C
    assert rows_per_w % _C == 0 and _C % 2 == 0
    n_trips = n_chunks // _NBUF
    n_tail = n_chunks - n_trips * _NBUF
    if n_tail < 2:
        n_trips -= 1
        n_tail += _NBUF
    assert n_trips >= 1 and n_tail >= 2

    def body(idx_hbm, word_hbm, pos_hbm, gamma_hbm, beta_hbm, out_hbm,
             idx0, idx1, idx2, rows0, rows1, rows2, sb0, sb1, sb2,
             pos_v, gamma_v, beta_v,
             gsem0, gsem1, gsem2, ssem0, ssem1, ssem2, isem0, isem1, isem2):
        idx_b = (idx0, idx1, idx2)
        rows_b = (rows0, rows1, rows2)
        sb_b = (sb0, sb1, sb2)
        gsem_b = (gsem0, gsem1, gsem2)
        ssem_b = (ssem0, ssem1, ssem2)
        isem_b = (isem0, isem1, isem2)

        wid = lax.axis_index("s") * _NC + lax.axis_index("c")
        base = wid * rows_per_w
        pltpu.sync_copy(pos_hbm, pos_v)
        pltpu.sync_copy(gamma_hbm, gamma_v)
        pltpu.sync_copy(beta_hbm, beta_v)
        g_regs = [gamma_v[pl.ds(16 * k, 16)] for k in range(_NREG)]
        b_regs = [beta_v[pl.ds(16 * k, 16)] for k in range(_NREG)]
        lanes = lax.iota(jnp.int32, 16)
        perms = [lanes ^ d for d in (1, 2, 4, 8)]

        def fire_idx(slot, g):
            cbase = row0 + base + g * _C
            pltpu.async_copy(idx_hbm.at[pl.ds(lax.div(cbase, _IG), _G)],
                             idx_b[slot], isem_b[slot])

        def wait_idx(slot):
            pltpu.make_async_copy(idx_hbm.at[pl.ds(0, _G)], idx_b[slot],
                                  isem_b[slot]).wait()

        def fire_gather(buf, g):
            for j in range(_G):
                pltpu.async_copy(word_hbm.at[idx_b[buf].at[j]],
                                 rows_b[buf].at[pl.ds(j * _IG, _IG)],
                                 gsem_b[buf])

        def wait_gather(buf):
            pltpu.make_async_copy(word_hbm.at[pl.ds(0, _C)], rows_b[buf],
                                  gsem_b[buf]).wait()

        def fire_store(buf, g):
            s128 = lax.div(base + g * _C, 2)
            pltpu.async_copy(sb_b[buf], out_hbm.at[pl.ds(s128, _C // 2)],
                             ssem_b[buf])

        def wait_store(buf):
            pltpu.make_async_copy(sb_b[buf], out_hbm.at[pl.ds(0, _C // 2)],
                                  ssem_b[buf]).wait()

        def compute(buf, cbase):
            rows_v = rows_b[buf]
            sb_v = sb_b[buf]
            off = lax.rem(row0 + cbase, seq_len)

            @plsc.parallel_loop(0, _C, unroll=4)
            def row_body(r):
                p = r + off
                for _ in range((_C + seq_len - 1) // seq_len + 1):
                    p = jnp.where(p >= seq_len, p - seq_len, p)
                q = r >> 1
                h0 = (r & 1) * _HID
                s = [rows_v[r, pl.ds(16 * k, 16)]
                     + pos_v[p, pl.ds(16 * k, 16)] for k in range(_NREG)]
                t = (s[0] + s[1]) + (s[2] + s[3])
                u = (s[0] * s[0] + s[1] * s[1]) + (s[2] * s[2] + s[3] * s[3])
                for perm in perms:
                    t = t + t.at[perm].get(mode="promise_in_bounds")
                    u = u + u.at[perm].get(mode="promise_in_bounds")
                m = t * (1.0 / _HID)
                var = jnp.maximum(u * (1.0 / _HID) - m * m, 0.0) + eps
                rs = _rsqrt16(var)
                for k in range(_NREG):
                    y = (s[k] - m) * rs
                    sb_v[q, pl.ds(h0 + 16 * k, 16)] = (y * g_regs[k]
                                                       + b_regs[k])

        fire_idx(0, 0)
        wait_idx(0)
        fire_gather(0, 0)
        fire_idx(1, 1)

        def trip_body(tt, carry):
            c0 = _NBUF * tt
            for par in range(_NBUF):
                nb = (par + 1) % _NBUF
                ns = (par + 2) % _NBUF
                wait_idx(nb)
                fire_gather(nb, c0 + par + 1)
                fire_idx(ns, c0 + par + 2)
                wait_gather(par)
                @pl.when(c0 + par >= _NBUF)
                def _():
                    wait_store(par)
                compute(par, base + (c0 + par) * _C)
                fire_store(par, c0 + par)
            return carry

        lax.fori_loop(0, n_trips, trip_body, 0)

        for c in range(n_trips * _NBUF, n_chunks):
            buf = c % _NBUF
            nb = (c + 1) % _NBUF
            if c + 1 < n_chunks:
                wait_idx(nb)
                fire_gather(nb, c + 1)
            if c + 2 < n_chunks:
                fire_idx((c + 2) % _NBUF, c + 2)
            wait_gather(buf)
            if c >= _NBUF:
                wait_store(buf)
            compute(buf, base + c * _C)
            fire_store(buf, c)

        for c in range(n_chunks - _NBUF, n_chunks):
            wait_store(c % _NBUF)

    return body


_NSPLIT = 1


def kernel(input_id, word_table, pos_table, gamma, beta):
    b, seq_len = input_id.shape
    n_rows = b * seq_len
    idx2d = input_id.reshape(n_rows // _IG, _IG).astype(jnp.int32)
    pos = pos_table[:seq_len]
    mesh = plsc.VectorSubcoreMesh(core_axis_name="c", subcore_axis_name="s")
    n_half = n_rows // _NSPLIT
    b_half = b // _NSPLIT
    parts = []
    for h in range(_NSPLIT):
        fn = functools.partial(
            pl.kernel,
            mesh=mesh,
            compiler_params=pltpu.CompilerParams(use_tc_tiling_on_sc=False),
            out_type=jax.ShapeDtypeStruct((n_half // 2, 2 * _HID),
                                          jnp.float32),
            scratch_types=(
                [pltpu.VMEM((_G, _IG), jnp.int32)] * _NBUF
                + [pltpu.VMEM((_C, _HID), jnp.float32)] * _NBUF
                + [pltpu.VMEM((_C // 2, 2 * _HID), jnp.float32)] * _NBUF
                + [pltpu.VMEM((seq_len, _HID), jnp.float32),
                   pltpu.VMEM((_HID,), jnp.float32),
                   pltpu.VMEM((_HID,), jnp.float32)]
                + [pltpu.SemaphoreType.DMA] * (3 * _NBUF)
            ),
            name=f"emb_ln_part{h}",
        )(_make_body(h * n_half, n_half, seq_len, 1e-12))
        parts.append(fn(idx2d, word_table, pos, gamma, beta)
                     .reshape(b_half, seq_len, _HID))
    if len(parts) == 1:
        return parts[0]
    return jnp.concatenate(parts, axis=0)
